# baseline (speedup 1.0000x reference)
import functools

import jax
import jax.numpy as jnp
from jax import lax
from jax.experimental import pallas as pl
from jax.experimental.pallas import tpu as pltpu
from jax.experimental.pallas import tpu_sc as plsc

_B = 1024
_S = 200
_D = 64
_NC = 2
_NS = 16
_NW = _NC * _NS
_ROWS_PER_W = _B * _S // _NW
_SEQ_PER_W = _ROWS_PER_W // _S
_CS = 2
_CHUNK = _CS * _S
_NCH = _ROWS_PER_W // _CHUNK
_LANES = 16
_VPR = _D // _LANES


_NBUF = 3


def _emb_body(x_hbm, table_hbm, pe_hbm, out_hbm,
              idx_v, pe_v, buf0, buf1, buf2,
              gsem0, gsem1, gsem2, ssem0, ssem1, ssem2):
    wid = lax.axis_index("s") * _NC + lax.axis_index("c")
    base = wid * _ROWS_PER_W

    pltpu.sync_copy(x_hbm.at[pl.ds(base, _ROWS_PER_W)], idx_v)
    pltpu.sync_copy(pe_hbm, pe_v)

    bufs = (buf0, buf1, buf2)
    gsems = (gsem0, gsem1, gsem2)
    ssems = (ssem0, ssem1, ssem2)

    def gather_cp(c):
        return pltpu.make_async_copy(
            table_hbm.at[idx_v.at[pl.ds(c * _CHUNK, _CHUNK)]],
            bufs[c % _NBUF], gsems[c % _NBUF])

    def store_cp(c):
        return pltpu.make_async_copy(
            bufs[c % _NBUF], out_hbm.at[pl.ds(base + c * _CHUNK, _CHUNK)],
            ssems[c % _NBUF])

    def add_pe(c):
        b = bufs[c % _NBUF]

        def seq_body(s, _):
            for c2 in range(_CS):
                r = c2 * _S + s
                for k in range(_VPR):
                    sl = pl.ds(k * _LANES, _LANES)
                    b[r, sl] = b[r, sl] + pe_v[s, sl]
            return 0

        lax.fori_loop(0, _S, seq_body, 0, unroll=2)

    gather_cp(0).start()
    gather_cp(1).start()
    for c in range(_NCH):
        gather_cp(c).wait()
        add_pe(c)
        store_cp(c).start()
        if c + 2 < _NCH:
            if c >= 1:
                store_cp(c - 1).wait()
            gather_cp(c + 2).start()
    store_cp(_NCH - 2).wait()
    store_cp(_NCH - 1).wait()


def _emb_call(x_flat, table, pe):
    mesh = plsc.VectorSubcoreMesh(
        core_axis_name="c", subcore_axis_name="s",
        num_cores=_NC, num_subcores=_NS)
    return pl.kernel(
        _emb_body,
        out_type=jax.ShapeDtypeStruct((_B * _S, _D), jnp.float32),
        mesh=mesh,
        compiler_params=pltpu.CompilerParams(use_tc_tiling_on_sc=False),
        scratch_types=[
            pltpu.VMEM((_ROWS_PER_W,), jnp.int32),
            pltpu.VMEM((_S, _D), jnp.float32),
            pltpu.VMEM((_CHUNK, _D), jnp.float32),
            pltpu.VMEM((_CHUNK, _D), jnp.float32),
            pltpu.VMEM((_CHUNK, _D), jnp.float32),
            pltpu.SemaphoreType.DMA,
            pltpu.SemaphoreType.DMA,
            pltpu.SemaphoreType.DMA,
            pltpu.SemaphoreType.DMA,
            pltpu.SemaphoreType.DMA,
            pltpu.SemaphoreType.DMA,
        ],
    )(x_flat, table, pe)


def kernel(x, table, pe):
    xf = x.reshape(-1).astype(jnp.int32)
    pe_s = pe[: x.shape[1]]
    out = _emb_call(xf, table, pe_s)
    return out.reshape(x.shape[0], x.shape[1], _D)
